# baseline (speedup 1.0000x reference)
import functools

import jax
import jax.numpy as jnp
from jax import lax
from jax.experimental import pallas as pl
from jax.experimental.pallas import tpu as pltpu
from jax.experimental.pallas import tpu_sc as plsc

_NC = 2
_NS = 16
_NW = _NC * _NS
_LANES = 16
_CHUNK = 128


def _build_sc_kernel(N, L, H, P, n_pat):
    assert N % (_NW * _CHUNK) == 0
    rows_per_w = N // _NW
    assert rows_per_w % L == 0
    n_chunks = rows_per_w // _CHUNK

    mesh = plsc.VectorSubcoreMesh(core_axis_name="c", subcore_axis_name="s")

    @functools.partial(
        pl.kernel,
        out_type=jax.ShapeDtypeStruct((N, H), jnp.float32),
        mesh=mesh,
        scratch_types=[
            pltpu.VMEM((3, 2, _CHUNK), jnp.int32),
            pltpu.VMEM((4, _CHUNK, H), jnp.float32),
            pltpu.VMEM((n_pat, _CHUNK), jnp.int32),
            pltpu.VMEM((L, H), jnp.float32),
            pltpu.VMEM((H,), jnp.float32),
            pltpu.VMEM_SHARED((L, H), jnp.float32),
            pltpu.VMEM_SHARED((P, H), jnp.float32),
            pltpu.SemaphoreType.DMA,
            pltpu.SemaphoreType.DMA,
            pltpu.SemaphoreType.DMA,
            pltpu.SemaphoreType.DMA,
            pltpu.SemaphoreType.DMA,
        ],
    )
    def sc_embed(ids_hbm, pidx_hbm, word_hbm, pos_hbm, tt_hbm, spk_hbm,
                 out_hbm, idx_v, obuf, pidx_v, posbuf, ttbuf,
                 pos_sp, spk_sp, sem_i, sem_w, sem_s, sem_p, sem_o):
        wid = lax.axis_index("s") * _NC + lax.axis_index("c")
        wbase = wid * rows_per_w

        @pl.when(lax.axis_index("s") == 0)
        def _():
            pltpu.sync_copy(spk_hbm, spk_sp)
            pltpu.sync_copy(pos_hbm.at[pl.ds(0, L)], posbuf)
            pltpu.sync_copy(tt_hbm.at[0], ttbuf)

            @plsc.parallel_loop(0, L, unroll=2)
            def _(r):
                for c in range(H // _LANES):
                    sl = pl.ds(c * _LANES, _LANES)
                    posbuf[r, sl] = posbuf[r, sl] + ttbuf[sl]

            pltpu.sync_copy(posbuf, pos_sp)

        pltpu.sync_copy(pidx_hbm, pidx_v)

        plsc.subcore_barrier()

        def fetch_ids(ck):
            si = lax.rem(ck, 3)
            rowbase = wbase + ck * _CHUNK
            pltpu.async_copy(ids_hbm.at[:, pl.ds(rowbase, _CHUNK)],
                             idx_v.at[si], sem_i)

        def wait_ids():
            pltpu.make_async_copy(ids_hbm.at[:, pl.ds(0, _CHUNK)],
                                  idx_v.at[0], sem_i).wait()

        def issue_word(ck):
            so = lax.rem(ck, 4)
            si = lax.rem(ck, 3)
            pltpu.async_copy(word_hbm.at[idx_v.at[si, 0]], obuf.at[so],
                             sem_w)

        def wait_word():
            pltpu.make_async_copy(word_hbm.at[idx_v.at[0, 0]], obuf.at[0],
                                  sem_w).wait()

        def issue_spk_add(ck):
            so = lax.rem(ck, 4)
            si = lax.rem(ck, 3)
            pltpu.async_copy(spk_sp.at[idx_v.at[si, 1]], obuf.at[so],
                             sem_s, add=True)

        def wait_spk_add():
            pltpu.make_async_copy(spk_sp.at[idx_v.at[0, 1]], obuf.at[0],
                                  sem_s).wait()

        def issue_pos_add(ck):
            so = lax.rem(ck, 4)
            p = lax.rem(ck, n_pat)
            pltpu.async_copy(pos_sp.at[pidx_v.at[p]], obuf.at[so],
                             sem_p, add=True)

        def wait_pos_add():
            pltpu.make_async_copy(pos_sp.at[pidx_v.at[0]], obuf.at[0],
                                  sem_p).wait()

        def issue_out(ck):
            so = lax.rem(ck, 4)
            rowbase = wbase + ck * _CHUNK
            pltpu.async_copy(obuf.at[so],
                             out_hbm.at[pl.ds(rowbase, _CHUNK)], sem_o)

        def wait_out():
            pltpu.make_async_copy(obuf.at[0], out_hbm.at[pl.ds(0, _CHUNK)],
                                  sem_o).wait()

        fetch_ids(0)
        wait_ids()
        issue_word(0)
        fetch_ids(1)
        wait_ids()
        issue_word(1)
        fetch_ids(2)

        @pl.loop(0, n_chunks)
        def _(ck):
            @pl.when(ck >= 1)
            def _():
                wait_pos_add()
                issue_out(ck - 1)

            wait_word()
            issue_spk_add(ck)

            @pl.when(ck + 2 < n_chunks)
            def _():
                wait_ids()

                @pl.when(ck >= 2)
                def _():
                    wait_out()

                issue_word(ck + 2)

            wait_spk_add()

            @pl.when(ck + 3 < n_chunks)
            def _():
                fetch_ids(ck + 3)

            issue_pos_add(ck)

        wait_pos_add()
        issue_out(n_chunks - 1)
        for _ in range(4):
            wait_out()

    return sc_embed


def kernel(input_ids, speaker_ids, word_table, position_table,
           token_type_table, speaker_table):
    B, L = input_ids.shape
    V, H = word_table.shape
    P = speaker_table.shape[0]
    N = B * L
    import math
    n_pat = math.lcm(_CHUNK, L) // _CHUNK
    sc = _build_sc_kernel(N, L, H, P, n_pat)
    ids = jnp.stack([input_ids.reshape(N).astype(jnp.int32),
                     speaker_ids.reshape(N).astype(jnp.int32)])
    pidx = (jnp.arange(n_pat * _CHUNK, dtype=jnp.int32) % L).reshape(
        n_pat, _CHUNK)
    out = sc(ids, pidx, word_table, position_table, token_type_table,
             speaker_table)
    return out.reshape(B, L, H)
